# baseline (speedup 1.0000x reference)
import jax
import jax.numpy as jnp
from jax import lax
from jax.experimental import pallas as pl
from jax.experimental.pallas import tpu as pltpu
from jax.experimental.pallas import tpu_sc as plsc

N_TOKENS = 16384
D = 1000
VOCAB = 1000
_LANES = 16
_TB = 2048
_NTB = N_TOKENS // _TB
_GROUPS = D // 8
_BIG = 4
_SMALL = 3
_NBIG = 29

_info = plsc.get_sparse_core_info()
_NC = _info.num_cores
_NS = _info.num_subcores
_NW = _NC * _NS


def _body(idx_hbm, tabT_hbm, outT_hbm, x_v, tab_v, rowbuf_v, wsems):
    c = lax.axis_index("c")
    s = lax.axis_index("s")
    wid = s * _NC + c

    pltpu.sync_copy(idx_hbm, x_v)

    def do_stripe(row_start, n_groups):
        row_start = pl.multiple_of(row_start, 8)
        n_rows = n_groups * 8
        pltpu.sync_copy(tabT_hbm.at[pl.ds(row_start, n_rows)],
                        tab_v.at[pl.ds(0, n_rows)])

        writes = [None, None]

        def fill(rg, tb, buf):
            def tok_body(t, carry):
                xv = x_v[pl.ds(tb * _TB + t * _LANES, _LANES)]
                for jj in range(8):
                    jvec = jnp.full((_LANES,), rg * 8 + jj, jnp.int32)
                    vals = plsc.load_gather(tab_v, [jvec, xv])
                    rowbuf_v[buf, jj, pl.ds(t * _LANES, _LANES)] = vals
                return carry
            lax.fori_loop(0, _TB // _LANES, tok_body, 0)

        step = 0
        for rg in range(n_groups):
            for tb in range(_NTB):
                buf = step % 2
                if writes[buf] is not None:
                    writes[buf].wait()
                fill(rg, tb, buf)
                writes[buf] = pltpu.async_copy(
                    rowbuf_v.at[buf],
                    outT_hbm.at[pl.ds(row_start + rg * 8, 8),
                                pl.ds(tb * _TB, _TB)],
                    wsems.at[buf])
                step += 1
        for b in range(2):
            if writes[b] is not None:
                writes[b].wait()

    @pl.when(wid < _NBIG)
    def _():
        do_stripe(wid * (_BIG * 8), _BIG)

    @pl.when(wid >= _NBIG)
    def _():
        do_stripe(_NBIG * _BIG * 8 + (wid - _NBIG) * (_SMALL * 8), _SMALL)


def kernel(x, table):
    idx = x.astype(jnp.int32)
    tabT = table.T
    gather = pl.kernel(
        _body,
        out_type=jax.ShapeDtypeStruct((D, N_TOKENS), jnp.float32),
        mesh=plsc.VectorSubcoreMesh(core_axis_name="c", subcore_axis_name="s"),
        scratch_types=[
            pltpu.VMEM((N_TOKENS,), jnp.int32),
            pltpu.VMEM((_BIG * 8, VOCAB), jnp.float32),
            pltpu.VMEM((2, 8, _TB), jnp.float32),
            pltpu.SemaphoreType.DMA((2,)),
        ],
        compiler_params=pltpu.CompilerParams(needs_layout_passes=False),
    )
    return gather(idx, tabT).T


# speedup vs baseline: 1.0532x; 1.0532x over previous
import jax
import jax.numpy as jnp
from jax import lax
from jax.experimental import pallas as pl
from jax.experimental.pallas import tpu as pltpu
from jax.experimental.pallas import tpu_sc as plsc

N_TOKENS = 16384
D = 1000
VOCAB = 1000
ROW_PAD = 1024
_LANES = 16
_TB = 2048
_NTB = N_TOKENS // _TB
_BIG = 4
_SMALL = 3
_NBIG = 29

_info = plsc.get_sparse_core_info()
_NC = _info.num_cores
_NS = _info.num_subcores
_NW = _NC * _NS


def _body(idx_hbm, tabT_hbm, outT_hbm, x_v, tab_v, rowbuf_v, wsems):
    c = lax.axis_index("c")
    s = lax.axis_index("s")
    wid = s * _NC + c

    pltpu.sync_copy(idx_hbm, x_v)

    def do_stripe(row_start, n_groups):
        row_start = pl.multiple_of(row_start, 8)
        n_rows = n_groups * 8
        pltpu.sync_copy(tabT_hbm.at[pl.ds(row_start * ROW_PAD,
                                          n_rows * ROW_PAD)],
                        tab_v.at[pl.ds(0, n_rows * ROW_PAD)])

        writes = [None, None]

        def fill(rg, tb, buf):
            row_refs = [tab_v.at[pl.ds((rg * 8 + jj) * ROW_PAD, ROW_PAD)]
                        for jj in range(8)]

            def tok_body(t, carry):
                xv = x_v[pl.ds(tb * _TB + t * _LANES, _LANES)]
                for jj in range(8):
                    vals = plsc.load_gather(row_refs[jj], [xv])
                    rowbuf_v[buf, jj, pl.ds(t * _LANES, _LANES)] = vals
                return carry
            lax.fori_loop(0, _TB // _LANES, tok_body, 0)

        step = 0
        for rg in range(n_groups):
            for tb in range(_NTB):
                buf = step % 2
                if writes[buf] is not None:
                    writes[buf].wait()
                fill(rg, tb, buf)
                writes[buf] = pltpu.async_copy(
                    rowbuf_v.at[buf],
                    outT_hbm.at[pl.ds(row_start + rg * 8, 8),
                                pl.ds(tb * _TB, _TB)],
                    wsems.at[buf])
                step += 1
        for b in range(2):
            if writes[b] is not None:
                writes[b].wait()

    @pl.when(wid < _NBIG)
    def _():
        do_stripe(wid * (_BIG * 8), _BIG)

    @pl.when(wid >= _NBIG)
    def _():
        do_stripe(_NBIG * _BIG * 8 + (wid - _NBIG) * (_SMALL * 8), _SMALL)


def kernel(x, table):
    idx = x.astype(jnp.int32)
    tabT_flat = jnp.pad(table.T, ((0, 0), (0, ROW_PAD - VOCAB))).reshape(-1)
    gather = pl.kernel(
        _body,
        out_type=jax.ShapeDtypeStruct((D, N_TOKENS), jnp.float32),
        mesh=plsc.VectorSubcoreMesh(core_axis_name="c", subcore_axis_name="s"),
        scratch_types=[
            pltpu.VMEM((N_TOKENS,), jnp.int32),
            pltpu.VMEM((_BIG * 8 * ROW_PAD,), jnp.float32),
            pltpu.VMEM((2, 8, _TB), jnp.float32),
            pltpu.SemaphoreType.DMA((2,)),
        ],
        compiler_params=pltpu.CompilerParams(needs_layout_passes=False),
    )
    return gather(idx, tabT_flat).T


# speedup vs baseline: 2.8148x; 2.6727x over previous
import jax
import jax.numpy as jnp
from jax import lax
from jax.experimental import pallas as pl
from jax.experimental.pallas import tpu as pltpu
from jax.experimental.pallas import tpu_sc as plsc

N_TOKENS = 16384
D = 1000
VOCAB = 1000
ROW_PAD = 1024
_LANES = 16
_TB = 2048
_NTB = N_TOKENS // _TB
_BIG = 4
_SMALL = 3
_NBIG = 29

_info = plsc.get_sparse_core_info()
_NC = _info.num_cores
_NS = _info.num_subcores
_NW = _NC * _NS


def _body(idx_hbm, tabT_hbm, outT_hbm, x_v, tab_v, rowbuf_v, wsems):
    c = lax.axis_index("c")
    s = lax.axis_index("s")
    wid = s * _NC + c

    pltpu.sync_copy(idx_hbm, x_v)

    def do_stripe(row_start, n_groups):
        row_start = pl.multiple_of(row_start, 8)
        n_rows = n_groups * 8
        pltpu.sync_copy(tabT_hbm.at[pl.ds(row_start * ROW_PAD,
                                          n_rows * ROW_PAD)],
                        tab_v.at[pl.ds(0, n_rows * ROW_PAD)])

        writes = [None, None]

        def fill(rg, tb, buf):
            row_refs = [tab_v.at[pl.ds((rg * 8 + jj) * ROW_PAD, ROW_PAD)]
                        for jj in range(8)]

            @plsc.parallel_loop(0, _TB // _LANES, unroll=2)
            def tok_body(t):
                xv = x_v[pl.ds(tb * _TB + t * _LANES, _LANES)]
                vals = [plsc.load_gather(row_refs[jj], [xv])
                        for jj in range(8)]
                for jj in range(8):
                    rowbuf_v[buf, jj, pl.ds(t * _LANES, _LANES)] = vals[jj]

        step = 0
        for rg in range(n_groups):
            for tb in range(_NTB):
                buf = step % 2
                if writes[buf] is not None:
                    writes[buf].wait()
                fill(rg, tb, buf)
                writes[buf] = pltpu.async_copy(
                    rowbuf_v.at[buf],
                    outT_hbm.at[pl.ds(row_start + rg * 8, 8),
                                pl.ds(tb * _TB, _TB)],
                    wsems.at[buf])
                step += 1
        for b in range(2):
            if writes[b] is not None:
                writes[b].wait()

    @pl.when(wid < _NBIG)
    def _():
        do_stripe(wid * (_BIG * 8), _BIG)

    @pl.when(wid >= _NBIG)
    def _():
        do_stripe(_NBIG * _BIG * 8 + (wid - _NBIG) * (_SMALL * 8), _SMALL)


def kernel(x, table):
    idx = x.astype(jnp.int32)
    tabT_flat = jnp.pad(table.T, ((0, 0), (0, ROW_PAD - VOCAB))).reshape(-1)
    gather = pl.kernel(
        _body,
        out_type=jax.ShapeDtypeStruct((D, N_TOKENS), jnp.float32),
        mesh=plsc.VectorSubcoreMesh(core_axis_name="c", subcore_axis_name="s"),
        scratch_types=[
            pltpu.VMEM((N_TOKENS,), jnp.int32),
            pltpu.VMEM((_BIG * 8 * ROW_PAD,), jnp.float32),
            pltpu.VMEM((2, 8, _TB), jnp.float32),
            pltpu.SemaphoreType.DMA((2,)),
        ],
        compiler_params=pltpu.CompilerParams(needs_layout_passes=False),
    )
    return gather(idx, tabT_flat).T
